# baseline (speedup 1.0000x reference)
import functools

import jax
import jax.numpy as jnp
from jax import lax
from jax.experimental import pallas as pl
from jax.experimental.pallas import tpu as pltpu
from jax.experimental.pallas import tpu_sc as plsc

NC = 2
NS = 16
LANES = 16
CHUNK = 80
NBUF = 4


def _matmul_body(x_ref, w_ref, o_ref):
    o_ref[...] = jnp.dot(x_ref[...], w_ref[...],
                         preferred_element_type=jnp.float32)


def _combine_body(a_ref, b_ref, bias_ref, o_ref):
    o_ref[...] = a_ref[0] + b_ref[0] + bias_ref[...]


def _make_spmm(n_nodes, d, per_tile, n_full):
    rows_per_tile = (n_nodes // NS) // 8 * 8
    leftover = n_nodes - rows_per_tile * NS
    assert leftover % 8 == 0
    zrows = rows_per_tile
    for cand in (16, 8):
        if rows_per_tile % cand == 0:
            zrows = cand
            break
    nz = rows_per_tile // zrows
    assert leftover <= zrows
    nv = d // LANES
    assert n_full % NBUF == 0 and n_full >= 2 * NBUF
    assert CHUNK % LANES == 0

    mesh = plsc.VectorSubcoreMesh(core_axis_name="c", subcore_axis_name="s")

    scratch = (
        [pltpu.VMEM((3 * CHUNK,), jnp.int32) for _ in range(NBUF)] +
        [pltpu.VMEM((CHUNK,), jnp.int32) for _ in range(NBUF)] +
        [pltpu.VMEM((CHUNK, d), jnp.float32) for _ in range(NBUF)] +
        [pltpu.VMEM((zrows, d), jnp.float32),
         pltpu.VMEM_SHARED((n_nodes, d), jnp.float32)] +
        [pltpu.SemaphoreType.DMA for _ in range(NBUF)] +
        [pltpu.SemaphoreType.DMA for _ in range(NBUF)] +
        [pltpu.SemaphoreType.DMA for _ in range(NBUF)]
    )

    def body(h_hbm, epk_hbm, out_hbm, *refs):
        epk = refs[0:NBUF]
        rv = refs[NBUF:2 * NBUF]
        rows = refs[2 * NBUF:3 * NBUF]
        zblk = refs[3 * NBUF]
        agg = refs[3 * NBUF + 1]
        sems = refs[3 * NBUF + 2:]
        sidx = sems[0:NBUF]
        sgat = sems[NBUF:2 * NBUF]
        ssc = sems[2 * NBUF:3 * NBUF]

        c = lax.axis_index("c")
        s = lax.axis_index("s")
        cb = (c * NS + s) * n_full

        def issue_idx(k, b):
            off = (cb + k) * 3 * CHUNK
            pltpu.async_copy(epk_hbm.at[pl.ds(off, 3 * CHUNK)], epk[b],
                             sidx[b])

        def wait_idx(k, b):
            off = (cb + k) * 3 * CHUNK
            pltpu.make_async_copy(epk_hbm.at[pl.ds(off, 3 * CHUNK)], epk[b],
                                  sidx[b]).wait()

        for b in range(NBUF):
            issue_idx(b, b)

        @pl.loop(0, zrows)
        def _(i):
            for j in range(nv):
                zblk[i, pl.ds(j * LANES, LANES)] = jnp.zeros(
                    (LANES,), jnp.float32)

        base_row = s * rows_per_tile
        for q in range(nz):
            pltpu.sync_copy(zblk, agg.at[pl.ds(base_row + q * zrows, zrows)])
        if leftover:
            @pl.when(s == NS - 1)
            def _():
                pltpu.sync_copy(
                    zblk.at[pl.ds(0, leftover)],
                    agg.at[pl.ds(rows_per_tile * NS, leftover)])
        plsc.subcore_barrier()

        @pl.loop(0, n_full, step=NBUF)
        def _(g4):
            for b in range(NBUF):
                k = g4 + b

                @pl.when(k >= NBUF)
                def _():
                    pltpu.make_async_copy(rows[b], agg.at[rv[b]],
                                          ssc[b]).wait()
                wait_idx(k, b)
                pltpu.async_copy(h_hbm.at[epk[b].at[pl.ds(0, CHUNK)]],
                                 rows[b], sgat[b])
            for b in range(NBUF):
                k = g4 + b
                pltpu.make_async_copy(h_hbm.at[epk[b].at[pl.ds(0, CHUNK)]],
                                      rows[b], sgat[b]).wait()

                @pl.loop(0, CHUNK // LANES)
                def _(q):
                    sl16 = pl.ds(q * LANES, LANES)
                    rv[b][sl16] = epk[b][pl.ds(CHUNK + q * LANES, LANES)]
                    wv16 = jax.lax.bitcast_convert_type(
                        epk[b][pl.ds(2 * CHUNK + q * LANES, LANES)],
                        jnp.float32)
                    for l in range(LANES):
                        wb = jnp.full((LANES,), wv16[l], dtype=jnp.float32)
                        e = q * LANES + l
                        for j in range(nv):
                            sl = pl.ds(j * LANES, LANES)
                            rows[b][e, sl] = rows[b][e, sl] * wb

                pltpu.async_copy(rows[b], agg.at[rv[b]], ssc[b], add=True)

                @pl.when(k + NBUF < n_full)
                def _():
                    issue_idx(k + NBUF, b)

        for b in range(NBUF):
            pltpu.make_async_copy(rows[b], agg.at[rv[b]], ssc[b]).wait()

        plsc.subcore_barrier()
        pltpu.sync_copy(agg.at[pl.ds(base_row, rows_per_tile)],
                        out_hbm.at[c, pl.ds(base_row, rows_per_tile)])
        if leftover:
            @pl.when(s == NS - 1)
            def _():
                pltpu.sync_copy(
                    agg.at[pl.ds(rows_per_tile * NS, leftover)],
                    out_hbm.at[c, pl.ds(rows_per_tile * NS, leftover)])

    return pl.kernel(
        body,
        out_type=jax.ShapeDtypeStruct((NC, n_nodes, d), jnp.float32),
        mesh=mesh,
        scratch_types=scratch,
    )


@jax.jit
def kernel(x, edge_index, edge_weight, weight, bias):
    n, d_in = x.shape
    d = weight.shape[1]
    n_edges = edge_weight.shape[0]

    blk = 1000 if n % 1000 == 0 else n
    nb = n // blk
    h = pl.pallas_call(
        _matmul_body,
        grid=(nb,),
        in_specs=[
            pl.BlockSpec((blk, d_in), lambda i: (i, 0)),
            pl.BlockSpec((d_in, d), lambda i: (0, 0)),
        ],
        out_specs=pl.BlockSpec((blk, d), lambda i: (i, 0)),
        out_shape=jax.ShapeDtypeStruct((n, d), jnp.float32),
    )(x, weight)

    q = CHUNK * NBUF * 2
    per_tile = -(-n_edges // (NC * NS * q)) * q
    n_full = per_tile // CHUNK
    total = per_tile * NC * NS

    ei = edge_index.astype(jnp.int32)
    ew = edge_weight.astype(jnp.float32)
    pad = total - n_edges
    if pad:
        spread = (jnp.arange(pad, dtype=jnp.int32) * 8) % n
        row = jnp.concatenate([ei[0], spread])
        col = jnp.concatenate([ei[1], spread])
        w = jnp.concatenate([ew, jnp.zeros((pad,), jnp.float32)])
    else:
        row, col, w = ei[0], ei[1], ew

    epk = jnp.concatenate([
        col.reshape(-1, CHUNK),
        row.reshape(-1, CHUNK),
        jax.lax.bitcast_convert_type(w, jnp.int32).reshape(-1, CHUNK),
    ], axis=1).reshape(-1)

    partials = _make_spmm(n, d, per_tile, n_full)(h, epk)

    out = pl.pallas_call(
        _combine_body,
        grid=(nb,),
        in_specs=[
            pl.BlockSpec((1, blk, d), lambda i: (0, i, 0)),
            pl.BlockSpec((1, blk, d), lambda i: (1, i, 0)),
            pl.BlockSpec((d,), lambda i: (0,)),
        ],
        out_specs=pl.BlockSpec((blk, d), lambda i: (i, 0)),
        out_shape=jax.ShapeDtypeStruct((n, d), jnp.float32),
    )(partials, partials, bias)
    return out


# speedup vs baseline: 1.0525x; 1.0525x over previous
import functools

import jax
import jax.numpy as jnp
from jax import lax
from jax.experimental import pallas as pl
from jax.experimental.pallas import tpu as pltpu
from jax.experimental.pallas import tpu_sc as plsc

NC = 2
NS = 16
LANES = 16
CHUNK = 64
NBUF = 4


def _matmul_body(x_ref, w_ref, o_ref):
    o_ref[...] = jnp.dot(x_ref[...], w_ref[...],
                         preferred_element_type=jnp.float32)


def _combine_body(a_ref, b_ref, bias_ref, o_ref):
    o_ref[...] = a_ref[0] + b_ref[0] + bias_ref[...]


def _make_spmm(n_nodes, d, per_tile, n_full):
    rows_per_tile = (n_nodes // NS) // 8 * 8
    leftover = n_nodes - rows_per_tile * NS
    assert leftover % 8 == 0
    zrows = rows_per_tile
    for cand in (16, 8):
        if rows_per_tile % cand == 0:
            zrows = cand
            break
    nz = rows_per_tile // zrows
    assert leftover <= zrows
    nv = d // LANES
    assert n_full % NBUF == 0 and n_full >= 2 * NBUF
    assert CHUNK % LANES == 0

    mesh = plsc.VectorSubcoreMesh(core_axis_name="c", subcore_axis_name="s")

    scratch = (
        [pltpu.VMEM((CHUNK,), jnp.int32) for _ in range(NBUF)] +
        [pltpu.VMEM((CHUNK,), jnp.int32) for _ in range(NBUF)] +
        [pltpu.VMEM((CHUNK,), jnp.float32) for _ in range(NBUF)] +
        [pltpu.VMEM((CHUNK,), jnp.int32) for _ in range(NBUF)] +
        [pltpu.VMEM((CHUNK, d), jnp.float32) for _ in range(NBUF)] +
        [pltpu.VMEM((zrows, d), jnp.float32),
         pltpu.VMEM_SHARED((n_nodes, d), jnp.float32)] +
        [pltpu.SemaphoreType.DMA for _ in range(NBUF)] +
        [pltpu.SemaphoreType.DMA for _ in range(NBUF)] +
        [pltpu.SemaphoreType.DMA for _ in range(NBUF)]
    )

    def body(h_hbm, col_hbm, row_hbm, w_hbm, out_hbm, *refs):
        colv = refs[0:NBUF]
        rowv = refs[NBUF:2 * NBUF]
        wv = refs[2 * NBUF:3 * NBUF]
        rv = refs[3 * NBUF:4 * NBUF]
        rows = refs[4 * NBUF:5 * NBUF]
        zblk = refs[5 * NBUF]
        agg = refs[5 * NBUF + 1]
        sems = refs[5 * NBUF + 2:]
        sidx = sems[0:NBUF]
        sgat = sems[NBUF:2 * NBUF]
        ssc = sems[2 * NBUF:3 * NBUF]

        c = lax.axis_index("c")
        s = lax.axis_index("s")
        eb = (c * NS + s) * per_tile

        def issue_idx(k, b):
            off = eb + k * CHUNK
            pltpu.async_copy(col_hbm.at[pl.ds(off, CHUNK)], colv[b], sidx[b])
            pltpu.async_copy(row_hbm.at[pl.ds(off, CHUNK)], rowv[b], sidx[b])
            pltpu.async_copy(w_hbm.at[pl.ds(off, CHUNK)], wv[b], sidx[b])

        def wait_idx(k, b):
            off = eb + k * CHUNK
            pltpu.make_async_copy(col_hbm.at[pl.ds(off, CHUNK)], colv[b],
                                  sidx[b]).wait()
            pltpu.make_async_copy(row_hbm.at[pl.ds(off, CHUNK)], rowv[b],
                                  sidx[b]).wait()
            pltpu.make_async_copy(w_hbm.at[pl.ds(off, CHUNK)], wv[b],
                                  sidx[b]).wait()

        for b in range(NBUF):
            issue_idx(b, b)

        @pl.loop(0, zrows)
        def _(i):
            for j in range(nv):
                zblk[i, pl.ds(j * LANES, LANES)] = jnp.zeros(
                    (LANES,), jnp.float32)

        base_row = s * rows_per_tile
        for q in range(nz):
            pltpu.sync_copy(zblk, agg.at[pl.ds(base_row + q * zrows, zrows)])
        if leftover:
            @pl.when(s == NS - 1)
            def _():
                pltpu.sync_copy(
                    zblk.at[pl.ds(0, leftover)],
                    agg.at[pl.ds(rows_per_tile * NS, leftover)])
        plsc.subcore_barrier()

        @pl.loop(0, n_full, step=NBUF)
        def _(g4):
            for b in range(NBUF):
                k = g4 + b

                @pl.when(k >= NBUF)
                def _():
                    pltpu.make_async_copy(rows[b], agg.at[rv[b]],
                                          ssc[b]).wait()
                wait_idx(k, b)
                pltpu.async_copy(h_hbm.at[colv[b]], rows[b], sgat[b])
            for b in range(NBUF):
                k = g4 + b
                pltpu.make_async_copy(h_hbm.at[colv[b]], rows[b],
                                      sgat[b]).wait()

                @pl.loop(0, CHUNK // LANES)
                def _(q):
                    sl16 = pl.ds(q * LANES, LANES)
                    rv[b][sl16] = rowv[b][sl16]
                    wv16 = wv[b][sl16]
                    for l in range(LANES):
                        wb = jnp.full((LANES,), wv16[l], dtype=jnp.float32)
                        e = q * LANES + l
                        for j in range(nv):
                            sl = pl.ds(j * LANES, LANES)
                            rows[b][e, sl] = rows[b][e, sl] * wb

                pltpu.async_copy(rows[b], agg.at[rv[b]], ssc[b], add=True)

                @pl.when(k + NBUF < n_full)
                def _():
                    issue_idx(k + NBUF, b)

        for b in range(NBUF):
            pltpu.make_async_copy(rows[b], agg.at[rv[b]], ssc[b]).wait()

        plsc.subcore_barrier()
        pltpu.sync_copy(agg.at[pl.ds(base_row, rows_per_tile)],
                        out_hbm.at[c, pl.ds(base_row, rows_per_tile)])
        if leftover:
            @pl.when(s == NS - 1)
            def _():
                pltpu.sync_copy(
                    agg.at[pl.ds(rows_per_tile * NS, leftover)],
                    out_hbm.at[c, pl.ds(rows_per_tile * NS, leftover)])

    return pl.kernel(
        body,
        out_type=jax.ShapeDtypeStruct((NC, n_nodes, d), jnp.float32),
        mesh=mesh,
        scratch_types=scratch,
    )


@jax.jit
def kernel(x, edge_index, edge_weight, weight, bias):
    n, d_in = x.shape
    d = weight.shape[1]
    n_edges = edge_weight.shape[0]

    blk = 1000 if n % 1000 == 0 else n
    nb = n // blk
    h = pl.pallas_call(
        _matmul_body,
        grid=(nb,),
        in_specs=[
            pl.BlockSpec((blk, d_in), lambda i: (i, 0)),
            pl.BlockSpec((d_in, d), lambda i: (0, 0)),
        ],
        out_specs=pl.BlockSpec((blk, d), lambda i: (i, 0)),
        out_shape=jax.ShapeDtypeStruct((n, d), jnp.float32),
    )(x, weight)

    q = CHUNK * NBUF * 2
    per_tile = -(-n_edges // (NC * NS * q)) * q
    n_full = per_tile // CHUNK
    total = per_tile * NC * NS

    ei = edge_index.astype(jnp.int32)
    ew = edge_weight.astype(jnp.float32)
    pad = total - n_edges
    if pad:
        spread = (jnp.arange(pad, dtype=jnp.int32) * 8) % n
        row = jnp.concatenate([ei[0], spread])
        col = jnp.concatenate([ei[1], spread])
        w = jnp.concatenate([ew, jnp.zeros((pad,), jnp.float32)])
    else:
        row, col, w = ei[0], ei[1], ew

    partials = _make_spmm(n, d, per_tile, n_full)(h, col, row, w)

    out = pl.pallas_call(
        _combine_body,
        grid=(nb,),
        in_specs=[
            pl.BlockSpec((1, blk, d), lambda i: (0, i, 0)),
            pl.BlockSpec((1, blk, d), lambda i: (1, i, 0)),
            pl.BlockSpec((d,), lambda i: (0,)),
        ],
        out_specs=pl.BlockSpec((blk, d), lambda i: (i, 0)),
        out_shape=jax.ShapeDtypeStruct((n, d), jnp.float32),
    )(partials, partials, bias)
    return out
